# Initial kernel scaffold; baseline (speedup 1.0000x reference)
import jax
import jax.numpy as jnp
from jax.experimental import pallas as pl


def kernel(x, edge_index, node_graph_ids, temp, phys, W1, b1, W2, b2, g1, be1, g2, be2, g3, be3, g4, be4, Wt, Wp, Wc1, bc1, Wc2, bc2, Wc3, bc3):
    raise NotImplementedError("write your pallas kernel here")


# speedup vs baseline: 1.0238x; 1.0238x over previous
import functools

import jax
import jax.numpy as jnp
from jax import lax
from jax.experimental import pallas as pl
from jax.experimental.pallas import tpu as pltpu

NN = 10000
NP = 10240
EE = 320000
DD = 128
BB = 64
NBLK = 10
BLK = 1024
EBLK = 1280
NEB = EE // EBLK
NHI = NP // 128

_F32 = jnp.float32


def _deg_body(sa_ref, sb_ref, da_ref, db_ref, os_ref, od_ref):
    i = pl.program_id(0)

    @pl.when(i == 0)
    def _():
        os_ref[...] = jnp.zeros((NHI, 128), _F32)
        od_ref[...] = jnp.zeros((NHI, 128), _F32)

    ihi = lax.broadcasted_iota(jnp.int32, (NHI, EBLK), 0)
    ilo = lax.broadcasted_iota(jnp.int32, (EBLK, 128), 1)

    def hist(a_ref, b_ref, o_ref):
        g = a_ref[0]
        gt = b_ref[0]
        ohhi = (ihi == (g >> 7)).astype(_F32)
        ohlo = (ilo == (gt & 127)).astype(_F32)
        o_ref[...] += jnp.dot(ohhi, ohlo, preferred_element_type=_F32)

    hist(sa_ref, sb_ref, os_ref)
    hist(da_ref, db_ref, od_ref)


def _deg_call(src, dst):
    sa = src.reshape(NEB, 1, EBLK)
    sb = src.reshape(NEB, EBLK, 1)
    da = dst.reshape(NEB, 1, EBLK)
    db = dst.reshape(NEB, EBLK, 1)
    outs = pl.pallas_call(
        _deg_body,
        grid=(NEB,),
        in_specs=[
            pl.BlockSpec((1, 1, EBLK), lambda i: (i, 0, 0)),
            pl.BlockSpec((1, EBLK, 1), lambda i: (i, 0, 0)),
            pl.BlockSpec((1, 1, EBLK), lambda i: (i, 0, 0)),
            pl.BlockSpec((1, EBLK, 1), lambda i: (i, 0, 0)),
        ],
        out_specs=[
            pl.BlockSpec((NHI, 128), lambda i: (0, 0)),
            pl.BlockSpec((NHI, 128), lambda i: (0, 0)),
        ],
        out_shape=[
            jax.ShapeDtypeStruct((NHI, 128), _F32),
            jax.ShapeDtypeStruct((NHI, 128), _F32),
        ],
    )(sa, sb, da, db)
    return outs[0].reshape(NP, 1), outs[1].reshape(NP, 1)


def _scale_body(x_ref, d_ref, o_ref):
    o_ref[...] = x_ref[...] * lax.rsqrt(jnp.maximum(d_ref[...], 1.0))


def _scale_call(x_p, dego):
    return pl.pallas_call(
        _scale_body,
        grid=(NBLK,),
        in_specs=[
            pl.BlockSpec((BLK, DD), lambda i: (i, 0)),
            pl.BlockSpec((BLK, 1), lambda i: (i, 0)),
        ],
        out_specs=pl.BlockSpec((BLK, DD), lambda i: (i, 0)),
        out_shape=jax.ShapeDtypeStruct((NP, DD), _F32),
    )(x_p, dego)


def _dense_body(p_ref, di_ref, do_ref, w_ref, b_ref, g_ref, be_ref, o_ref,
                *, scale_out):
    a = p_ref[...] * lax.rsqrt(jnp.maximum(di_ref[...], 1.0))
    y = jnp.dot(a, w_ref[...], preferred_element_type=_F32) + b_ref[...]
    mu = jnp.mean(y, axis=-1, keepdims=True)
    yc = y - mu
    var = jnp.mean(yc * yc, axis=-1, keepdims=True)
    h = jnp.maximum(yc * lax.rsqrt(var + 1e-5) * g_ref[...] + be_ref[...], 0.0)
    if scale_out:
        h = h * lax.rsqrt(jnp.maximum(do_ref[...], 1.0))
    o_ref[...] = h


def _dense_call(p, degi, dego, w, b, g, be, scale_out):
    return pl.pallas_call(
        functools.partial(_dense_body, scale_out=scale_out),
        grid=(NBLK,),
        in_specs=[
            pl.BlockSpec((BLK, DD), lambda i: (i, 0)),
            pl.BlockSpec((BLK, 1), lambda i: (i, 0)),
            pl.BlockSpec((BLK, 1), lambda i: (i, 0)),
            pl.BlockSpec((DD, DD), lambda i: (0, 0)),
            pl.BlockSpec((1, DD), lambda i: (0, 0)),
            pl.BlockSpec((1, DD), lambda i: (0, 0)),
            pl.BlockSpec((1, DD), lambda i: (0, 0)),
        ],
        out_specs=pl.BlockSpec((BLK, DD), lambda i: (i, 0)),
        out_shape=jax.ShapeDtypeStruct((NP, DD), _F32),
    )(p, degi, dego, w, b, g, be)


def _ln(y, g, b):
    mu = jnp.mean(y, axis=-1, keepdims=True)
    yc = y - mu
    var = jnp.mean(yc * yc, axis=-1, keepdims=True)
    return yc * lax.rsqrt(var + 1e-5) * g + b


def _l2n(v):
    n = jnp.sqrt(jnp.sum(v * v, axis=1, keepdims=True))
    return v / jnp.maximum(n, 1e-12)


def _head_body(
    h2_ref, gid_ref, maxp_ref, temp_ref, phys_ref, wt_ref, wp_ref,
    wc1_ref, bc1_ref, g3_ref, be3_ref, wc2_ref, bc2_ref, g4_ref, be4_ref,
    wc3_ref, bc3_ref, o_ref, macc, cacc,
):
    i = pl.program_id(0)

    @pl.when(i == 0)
    def _():
        macc[...] = jnp.zeros((BB, DD), _F32)
        cacc[...] = jnp.zeros((BB, DD), _F32)

    g = gid_ref[0]
    oh = (lax.broadcasted_iota(jnp.int32, (BB, BLK), 0) == g).astype(_F32)
    macc[...] += jnp.dot(oh, h2_ref[...], preferred_element_type=_F32)
    cacc[...] = cacc[...] + jnp.sum(oh, axis=1, keepdims=True)

    @pl.when(i == NBLK - 1)
    def _():
        mean = macc[...] / jnp.maximum(cacc[...], 1.0)
        mx = maxp_ref[...]
        temp = temp_ref[...]
        temp_f = jnp.maximum(
            jnp.dot(temp, wt_ref[...], preferred_element_type=_F32), 0.0
        )
        ph = jnp.concatenate([phys_ref[...], temp], axis=1)
        phys_f = jnp.maximum(
            jnp.dot(ph, wp_ref[...], preferred_element_type=_F32), 0.0
        )
        hg = jnp.concatenate([_l2n(mean), _l2n(mx), temp_f, phys_f], axis=1)
        o1 = jnp.maximum(
            _ln(jnp.dot(hg, wc1_ref[...], preferred_element_type=_F32)
                + bc1_ref[...], g3_ref[...], be3_ref[...]), 0.0)
        o2 = jnp.maximum(
            _ln(jnp.dot(o1, wc2_ref[...], preferred_element_type=_F32)
                + bc2_ref[...], g4_ref[...], be4_ref[...]), 0.0)
        o_ref[...] = (
            jnp.dot(o2, wc3_ref[...], preferred_element_type=_F32) + bc3_ref[...]
        )


def _head_call(h2, gids3, maxp, temp, phys, wt, wp, wc1, bc1, g3, be3,
               wc2, bc2, g4, be4, wc3, bc3):
    full = lambda shape: pl.BlockSpec(shape, lambda i: tuple(0 for _ in shape))
    return pl.pallas_call(
        _head_body,
        grid=(NBLK,),
        in_specs=[
            pl.BlockSpec((BLK, DD), lambda i: (i, 0)),
            pl.BlockSpec((1, 1, BLK), lambda i: (i, 0, 0)),
            full((BB, DD)),
            full((BB, 1)),
            full((BB, 8)),
            full((1, DD)),
            full((9, DD)),
            full((4 * DD, DD)),
            full((1, DD)),
            full((1, DD)),
            full((1, DD)),
            full((DD, DD)),
            full((1, DD)),
            full((1, DD)),
            full((1, DD)),
            full((DD, 16)),
            full((1, 16)),
        ],
        out_specs=full((BB, 16)),
        out_shape=jax.ShapeDtypeStruct((BB, 16), _F32),
        scratch_shapes=[
            pltpu.VMEM((BB, DD), _F32),
            pltpu.VMEM((BB, DD), _F32),
        ],
    )(h2, gids3, maxp, temp, phys, wt, wp, wc1, bc1, g3, be3,
      wc2, bc2, g4, be4, wc3, bc3)


def kernel(x, edge_index, node_graph_ids, temp, phys, W1, b1, W2, b2,
           g1, be1, g2, be2, g3, be3, g4, be4, Wt, Wp, Wc1, bc1, Wc2, bc2,
           Wc3, bc3):
    src = edge_index[0]
    dst = edge_index[1]
    x_p = jnp.zeros((NP, DD), _F32).at[:NN].set(x)
    gids_p = jnp.concatenate(
        [node_graph_ids, jnp.full((NP - NN,), BB, jnp.int32)]
    )
    gids3 = gids_p.reshape(NBLK, 1, BLK)

    r = lambda v: v.reshape(1, -1)

    dego, degi = _deg_call(src, dst)
    h0s = _scale_call(x_p, dego)
    p1 = jax.ops.segment_sum(h0s[src], dst, num_segments=NP)
    h1s = _dense_call(p1, degi, dego, W1, r(b1), r(g1), r(be1), scale_out=True)
    p2 = jax.ops.segment_sum(h1s[src], dst, num_segments=NP)
    h2 = _dense_call(p2, degi, dego, W2, r(b2), r(g2), r(be2), scale_out=False)
    mx = jax.ops.segment_max(h2[:NN], node_graph_ids, num_segments=BB,
                             indices_are_sorted=True)
    out = _head_call(h2, gids3, mx, temp, phys, Wt, Wp, Wc1, r(bc1),
                     r(g3), r(be3), Wc2, r(bc2), r(g4), r(be4), Wc3, r(bc3))
    return out
